# baseline (speedup 1.0000x reference)
import jax
import jax.numpy as jnp
from jax import lax
from jax.experimental import pallas as pl
from jax.experimental.pallas import tpu as pltpu
from jax.experimental.pallas import tpu_sc as plsc

_N = 10000
_D = 512
_E = 10000
_NLANE = 16
_R = 80
_NCHUNK = _N // _R
_G = 32
_U = 5
_LIST = _E + 2 * _G + _NLANE
_CNT = _E + 2 * _G
_NDSUB = _D // _NLANE
_NW = 32
_KMAX = (_NCHUNK + _NW - 1) // _NW


def _segmax_body(row_hbm, x_hbm, out_hbm, rows_v, el_a, el_b, out_c,
                 stage0, stage_a, stage_b, sem0, sem_a, sem_b):
    nc = 2
    wid = lax.axis_index("s") * nc + lax.axis_index("c")
    nk = (_NCHUNK - wid + _NW - 1) // _NW

    pltpu.sync_copy(row_hbm, rows_v.at[pl.ds(0, _E)])

    def chunk_lo(kk):
        return pl.multiple_of((kk * _NW + wid) * _R, 8)

    def scan_chunk(kk, el):
        lo = chunk_lo(kk)

        def scan80(i, count):
            base = i * (_U * _NLANE)
            ms, pcs = [], []
            for u in range(_U):
                v = rows_v[pl.ds(base + u * _NLANE, _NLANE)]
                m = (v >= lo) & (v < lo + _R)
                ms.append(m)
                pcs.append(plsc.all_reduce_population_count(m)[0])
            starts = []
            acc = count
            for u in range(_U):
                starts.append(acc)
                acc = acc + pcs[u]
            for u in range(_U):
                eids = lax.iota(jnp.int32, _NLANE) + (base + u * _NLANE)
                dest = starts[u] + plsc.cumsum(ms[u].astype(jnp.int32)) - 1
                plsc.store_scatter(el, [dest], eids, mask=ms[u])
            return acc
        count = lax.fori_loop(0, _E // (_U * _NLANE), scan80, jnp.int32(0))

        for t in range(_G // _NLANE):
            el[pl.ds(count + t * _NLANE, _NLANE)] = jnp.zeros(
                (_NLANE,), jnp.int32)
        el[pl.ds(_CNT, _NLANE)] = jnp.full((_NLANE,), count, jnp.int32)

    def issue(el, w, stage, sem):
        pltpu.make_async_copy(x_hbm.at[el.at[pl.ds(w * _G, _G)]], stage,
                              sem).start()

    def wait_wave(el, w, stage, sem):
        pltpu.make_async_copy(x_hbm.at[el.at[pl.ds(w * _G, _G)]], stage,
                              sem).wait()

    def maxacc(el, w, stage, count, lo):
        g0 = w * _G
        gn = jnp.minimum(count - g0, _G)

        def edge(g, _c):
            eid = el[pl.ds(g0 + g, _NLANE)][0]
            l = rows_v[pl.ds(eid, _NLANE)][0] - lo
            for d in range(_NDSUB):
                sl = pl.ds(d * _NLANE, _NLANE)
                out_c[l, sl] = jnp.maximum(out_c[l, sl], stage[g, sl])
            return _c
        lax.fori_loop(0, gn, edge, 0)

    def init_out():
        def init_row(r, c):
            for d in range(_NDSUB):
                out_c[r, pl.ds(d * _NLANE, _NLANE)] = jnp.full(
                    (_NLANE,), -jnp.inf, jnp.float32)
            return c
        lax.fori_loop(0, _R, init_row, 0)

    scan_chunk(0, el_a)
    _TESTA = False
    if not _TESTA:
        issue(el_a, 0, stage0, sem0)

    for kk in range(_KMAX):
        el = el_a if kk % 2 == 0 else el_b
        el_next = el_b if kk % 2 == 0 else el_a

        @pl.when(kk < nk)
        def _(kk=kk, el=el, el_next=el_next):
            @pl.when(kk + 1 < nk)
            def _():
                scan_chunk(kk + 1, el_next)
            init_out()

            lo = chunk_lo(kk)
            count = el[pl.ds(_CNT, _NLANE)][0]
            nwaves = (count + _G - 1) // _G

            if _TESTA:
                issue(el, 0, stage0, sem0)
            wait_wave(el, 0, stage0, sem0)

            @pl.when(nwaves > 1)
            def _():
                issue(el, 1, stage_a, sem_a)
            maxacc(el, 0, stage0, count, lo)

            def wavebody(w, _):
                odd = (w % 2) == 1

                @pl.when(odd)
                def _():
                    @pl.when(w + 1 < nwaves)
                    def _():
                        issue(el, w + 1, stage_b, sem_b)
                    wait_wave(el, w, stage_a, sem_a)
                    maxacc(el, w, stage_a, count, lo)

                @pl.when(jnp.logical_not(odd))
                def _():
                    @pl.when(w + 1 < nwaves)
                    def _():
                        issue(el, w + 1, stage_a, sem_a)
                    wait_wave(el, w, stage_b, sem_b)
                    maxacc(el, w, stage_b, count, lo)
                return 0
            lax.fori_loop(1, nwaves, wavebody, 0)

            pltpu.sync_copy(out_c, out_hbm.at[pl.ds(lo, _R)])

            if not _TESTA:
                @pl.when(kk + 1 < nk)
                def _():
                    issue(el_next, 0, stage0, sem0)


def _segmax(row, x):
    mesh = plsc.VectorSubcoreMesh(core_axis_name="c", subcore_axis_name="s")
    return pl.kernel(
        _segmax_body,
        out_type=jax.ShapeDtypeStruct((_N, _D), jnp.float32),
        mesh=mesh,
        compiler_params=pltpu.CompilerParams(needs_layout_passes=False,
                                             use_tc_tiling_on_sc=False),
        scratch_types=[
            pltpu.VMEM((_E + _NLANE,), jnp.int32),
            pltpu.VMEM((_LIST,), jnp.int32),
            pltpu.VMEM((_LIST,), jnp.int32),
            pltpu.VMEM((_R, _D), jnp.float32),
            pltpu.VMEM((_G, _D), jnp.float32),
            pltpu.VMEM((_G, _D), jnp.float32),
            pltpu.VMEM((_G, _D), jnp.float32),
            pltpu.SemaphoreType.DMA,
            pltpu.SemaphoreType.DMA,
            pltpu.SemaphoreType.DMA,
        ],
    )(row, x)


_BM = 1000


def _mm_body(seg_ref, x_ref, w_ref, b_ref, o_ref):
    seg = seg_ref[...]
    agg = jnp.where(seg == -jnp.inf, x_ref[...], seg)
    acc = lax.dot_general(agg, w_ref[...], (((1,), (1,)), ((), ())),
                          preferred_element_type=jnp.float32)
    o_ref[...] = acc + b_ref[...]


def _matmul(seg, x, W, b2d):
    return pl.pallas_call(
        _mm_body,
        grid=(_N // _BM,),
        in_specs=[
            pl.BlockSpec((_BM, _D), lambda i: (i, 0)),
            pl.BlockSpec((_BM, _D), lambda i: (i, 0)),
            pl.BlockSpec((_D, _D), lambda i: (0, 0)),
            pl.BlockSpec((1, _D), lambda i: (0, 0)),
        ],
        out_specs=pl.BlockSpec((_BM, _D), lambda i: (i, 0)),
        out_shape=jax.ShapeDtypeStruct((_N, _D), jnp.float32),
    )(seg, x, W, b2d)


@jax.jit
def kernel(x, edge_index, W, b):
    row = edge_index[0]
    seg = _segmax(row, x)
    return _matmul(seg, x, W, b.reshape(1, _D))


# speedup vs baseline: 1.2683x; 1.2683x over previous
import jax
import jax.numpy as jnp
from jax import lax
from jax.experimental import pallas as pl
from jax.experimental.pallas import tpu as pltpu
from jax.experimental.pallas import tpu_sc as plsc

_N = 10000
_D = 512
_E = 10000
_R = 80
_NCHUNK = _N // _R
_G = 32
_U = 5
_LIST = _E + 2 * _G
_NLANE = 16
_NDSUB = _D // _NLANE


def _segmax_body(row_hbm, x_hbm, out_hbm, rows_v, elist, dlist, out_c,
                 stage_a, stage_b, sem_a, sem_b):
    nc = 2
    wid = lax.axis_index("s") * nc + lax.axis_index("c")
    nw_total = nc * 16

    pltpu.sync_copy(row_hbm, rows_v)

    def do_chunk(k, _):
        chunk = k * nw_total + wid
        lo = pl.multiple_of(chunk * _R, 8)

        def init_row(r, c):
            for d in range(_NDSUB):
                out_c[r, pl.ds(d * _NLANE, _NLANE)] = jnp.full(
                    (_NLANE,), -jnp.inf, jnp.float32)
            return c
        lax.fori_loop(0, _R, init_row, 0)

        def scan80(i, count):
            base = i * (_U * _NLANE)
            vs, ms, pcs = [], [], []
            for u in range(_U):
                v = rows_v[pl.ds(base + u * _NLANE, _NLANE)]
                m = (v >= lo) & (v < lo + _R)
                vs.append(v)
                ms.append(m)
                pcs.append(plsc.all_reduce_population_count(m)[0])
            starts = []
            acc = count
            for u in range(_U):
                starts.append(acc)
                acc = acc + pcs[u]
            for u in range(_U):
                eids = lax.iota(jnp.int32, _NLANE) + (base + u * _NLANE)
                dest = starts[u] + plsc.cumsum(ms[u].astype(jnp.int32)) - 1
                plsc.store_scatter(elist, [dest], eids, mask=ms[u])
                plsc.store_scatter(dlist, [dest], vs[u] - lo, mask=ms[u])
            return acc
        count = lax.fori_loop(0, _E // (_U * _NLANE), scan80, jnp.int32(0))

        for t in range(_G // _NLANE):
            elist[pl.ds(count + t * _NLANE, _NLANE)] = jnp.zeros(
                (_NLANE,), jnp.int32)

        nwaves = (count + _G - 1) // _G

        def issue(w, stage, sem):
            pltpu.make_async_copy(x_hbm.at[elist.at[pl.ds(w * _G, _G)]],
                                  stage, sem).start()

        def process(w, stage, sem):
            g0 = w * _G
            pltpu.make_async_copy(x_hbm.at[elist.at[pl.ds(g0, _G)]], stage,
                                  sem).wait()
            gn = jnp.minimum(count - g0, _G)

            def edge(g, _c):
                l = dlist[pl.ds(g0 + g, _NLANE)][0]
                for d in range(_NDSUB):
                    sl = pl.ds(d * _NLANE, _NLANE)
                    out_c[l, sl] = jnp.maximum(out_c[l, sl], stage[g, sl])
                return _c
            lax.fori_loop(0, gn, edge, 0)

        @pl.when(nwaves > 0)
        def _():
            issue(0, stage_a, sem_a)

        def wave(w, _):
            even = (w % 2) == 0

            @pl.when(even)
            def _():
                @pl.when(w + 1 < nwaves)
                def _():
                    issue(w + 1, stage_b, sem_b)
                process(w, stage_a, sem_a)

            @pl.when(jnp.logical_not(even))
            def _():
                @pl.when(w + 1 < nwaves)
                def _():
                    issue(w + 1, stage_a, sem_a)
                process(w, stage_b, sem_b)
            return 0
        lax.fori_loop(0, nwaves, wave, 0)

        pltpu.sync_copy(out_c, out_hbm.at[pl.ds(lo, _R)])
        return 0

    nk = (_NCHUNK - wid + nw_total - 1) // nw_total
    lax.fori_loop(0, nk, do_chunk, 0)


def _segmax(row, x):
    mesh = plsc.VectorSubcoreMesh(core_axis_name="c", subcore_axis_name="s")
    return pl.kernel(
        _segmax_body,
        out_type=jax.ShapeDtypeStruct((_N, _D), jnp.float32),
        mesh=mesh,
        compiler_params=pltpu.CompilerParams(needs_layout_passes=False),
        scratch_types=[
            pltpu.VMEM((_E,), jnp.int32),
            pltpu.VMEM((_LIST,), jnp.int32),
            pltpu.VMEM((_LIST,), jnp.int32),
            pltpu.VMEM((_R, _D), jnp.float32),
            pltpu.VMEM((_G, _D), jnp.float32),
            pltpu.VMEM((_G, _D), jnp.float32),
            pltpu.SemaphoreType.DMA,
            pltpu.SemaphoreType.DMA,
        ],
    )(row, x)


_BM = 1000


def _mm_body(seg_ref, x_ref, w_ref, b_ref, o_ref):
    seg = seg_ref[...]
    agg = jnp.where(seg == -jnp.inf, x_ref[...], seg)
    acc = lax.dot_general(agg, w_ref[...], (((1,), (1,)), ((), ())),
                          preferred_element_type=jnp.float32)
    o_ref[...] = acc + b_ref[...]


def _matmul(seg, x, W, b2d):
    return pl.pallas_call(
        _mm_body,
        grid=(_N // _BM,),
        in_specs=[
            pl.BlockSpec((_BM, _D), lambda i: (i, 0)),
            pl.BlockSpec((_BM, _D), lambda i: (i, 0)),
            pl.BlockSpec((_D, _D), lambda i: (0, 0)),
            pl.BlockSpec((1, _D), lambda i: (0, 0)),
        ],
        out_specs=pl.BlockSpec((_BM, _D), lambda i: (i, 0)),
        out_shape=jax.ShapeDtypeStruct((_N, _D), jnp.float32),
    )(seg, x, W, b2d)


@jax.jit
def kernel(x, edge_index, W, b):
    row = edge_index[0]
    seg = _segmax(row, x)
    return _matmul(seg, x, W, b.reshape(1, _D))
